# baseline (speedup 1.0000x reference)
import functools

import jax
import jax.numpy as jnp
from jax import lax
from jax.experimental import pallas as pl
from jax.experimental.pallas import tpu as pltpu
from jax.experimental.pallas import tpu_sc as plsc

NC, NS = 2, 16
NW = NC * NS
CWP = 128
PADR = 16

_MESH = plsc.VectorSubcoreMesh(
    core_axis_name="c", subcore_axis_name="s", num_cores=NC, num_subcores=NS)


def _make_deg_kernel(N, CH):
    NP = N + PADR

    @functools.partial(
        pl.kernel, mesh=_MESH,
        out_type=jax.ShapeDtypeStruct((NC * 2 * NP,), jnp.float32),
        scratch_types=[
            pltpu.VMEM((CH, CWP), jnp.int32),
            pltpu.VMEM((CH, CWP), jnp.int32),
            pltpu.VMEM((CWP,), jnp.float32),
            pltpu.VMEM((NP,), jnp.float32),
            pltpu.VMEM_SHARED((NP,), jnp.float32),
            pltpu.VMEM_SHARED((NP,), jnp.float32),
            pltpu.SemaphoreType.DMA,
        ])
    def deg_kernel(src_hbm, dst_hbm, z_hbm, out_hbm,
                   src_v, dst_v, ones_v, tmp_v, acc_o, acc_i, sem):
        c = lax.axis_index("c")
        s = lax.axis_index("s")
        wid = c * NS + s
        pltpu.sync_copy(src_hbm.at[wid], src_v)
        pltpu.sync_copy(dst_hbm.at[wid], dst_v)
        for i in range(CWP // 16):
            ones_v[pl.ds(i * 16, 16)] = jnp.full((16,), 1.0, jnp.float32)

        @pl.when(s == 0)
        def _():
            pltpu.sync_copy(z_hbm, acc_o)

        @pl.when(s == 1)
        def _():
            pltpu.sync_copy(z_hbm, acc_i)

        plsc.subcore_barrier()

        LAG = 4

        def body(j, carry):
            @pl.when(j >= LAG)
            def _():
                pltpu.make_async_copy(
                    ones_v, acc_o.at[src_v.at[0]], sem).wait()
                pltpu.make_async_copy(
                    ones_v, acc_i.at[dst_v.at[0]], sem).wait()

            pltpu.async_copy(ones_v, acc_o.at[src_v.at[j]], sem, add=True)
            pltpu.async_copy(ones_v, acc_i.at[dst_v.at[j]], sem, add=True)
            return carry

        lax.fori_loop(0, CH, body, 0)
        for _ in range(LAG):
            pltpu.make_async_copy(ones_v, acc_o.at[src_v.at[0]], sem).wait()
            pltpu.make_async_copy(ones_v, acc_i.at[dst_v.at[0]], sem).wait()
        plsc.subcore_barrier()

        @pl.when(s == 0)
        def _():
            pltpu.sync_copy(acc_o, tmp_v)
            pltpu.sync_copy(
                tmp_v, out_hbm.at[pl.ds(pl.multiple_of(c * 2 * NP, 8), NP)])

        @pl.when(s == 1)
        def _():
            pltpu.sync_copy(acc_i, tmp_v)
            pltpu.sync_copy(
                tmp_v,
                out_hbm.at[pl.ds(pl.multiple_of(c * 2 * NP + NP, 8), NP)])

    return deg_kernel


def _make_agg_kernel(N, D, CH, B=None):
    NP = N + PADR

    PH = (CH + 1) // 2
    if B is None:
        out_type = jax.ShapeDtypeStruct((NC, N, D), jnp.float32)
        extra_scr = []
    else:
        BPW = B // NS
        KC = BPW // CWP
        out_type = (jax.ShapeDtypeStruct((2, NC, B, D), jnp.float32),
                    jax.ShapeDtypeStruct((NS, 2 * KC, CWP), jnp.float32))
        extra_scr = [pltpu.VMEM((2 * KC, CWP), jnp.float32)]

    @functools.partial(
        pl.kernel, mesh=_MESH,
        out_type=out_type,
        scratch_types=[
            pltpu.VMEM((PH, CWP), jnp.int32),
            pltpu.VMEM((PH, CWP), jnp.int32),
            pltpu.VMEM((2, CWP, D), jnp.float32),
            pltpu.VMEM((16, D), jnp.float32),
            pltpu.VMEM_SHARED((NP, D), jnp.float32),
            pltpu.SemaphoreType.DMA,
            pltpu.SemaphoreType.DMA,
        ] + extra_scr)
    def agg_kernel(h_hbm, src_hbm, dst_hbm, *rest):
        if B is None:
            (out_hbm, src_v, dst_v, rows_v, zb_v, acc, gsem, ssem) = rest
        else:
            (b_hbm, nd_hbm, e_hbm, ndo_hbm,
             src_v, dst_v, rows_v, zb_v, acc, gsem, ssem, ndb_v) = rest
        c = lax.axis_index("c")
        s = lax.axis_index("s")
        wid = c * NS + s

        def zrow(i, carry):
            for jj in range(D // 16):
                zb_v[i, pl.ds(jj * 16, 16)] = jnp.zeros((16,), jnp.float32)
            return carry

        lax.fori_loop(0, 16, zrow, 0)

        rpt = (NP // NS) & ~7
        last = NP - (NS - 1) * rpt
        nz = rpt // 16 + jnp.where(s == NS - 1, (last - rpt) // 16, 0)

        def zcopy(i, carry):
            r0 = pl.multiple_of(s * rpt + i * 16, 8)
            pltpu.sync_copy(zb_v, acc.at[pl.ds(r0, 16)])
            return carry

        lax.fori_loop(0, nz, zcopy, 0)
        plsc.subcore_barrier()

        def phase(base, nj):
            pltpu.sync_copy(src_hbm.at[wid].at[pl.ds(base, nj)],
                            src_v.at[pl.ds(0, nj)])
            pltpu.sync_copy(dst_hbm.at[wid].at[pl.ds(base, nj)],
                            dst_v.at[pl.ds(0, nj)])
            pltpu.async_copy(h_hbm.at[src_v.at[0]], rows_v.at[0], gsem)

            def body(j, carry):
                cur = lax.rem(j, 2)

                @pl.when(j >= 1)
                def _():
                    pltpu.make_async_copy(
                        rows_v.at[1 - cur], acc.at[dst_v.at[0]], ssem).wait()

                @pl.when(j + 1 < nj)
                def _():
                    pltpu.async_copy(
                        h_hbm.at[src_v.at[j + 1]], rows_v.at[1 - cur], gsem)

                pltpu.make_async_copy(
                    h_hbm.at[src_v.at[j]], rows_v.at[cur], gsem).wait()
                pltpu.async_copy(rows_v.at[cur], acc.at[dst_v.at[j]], ssem,
                                 add=True)
                return carry

            lax.fori_loop(0, nj, body, 0)
            pltpu.make_async_copy(
                rows_v.at[0], acc.at[dst_v.at[0]], ssem).wait()

        phase(0, PH)
        phase(PH, CH - PH)
        plsc.subcore_barrier()

        if B is None:
            wpt = (N // NS) & ~7
            wlast = N - (NS - 1) * wpt
            row0 = pl.multiple_of(s * wpt, 8)

            @pl.when(s < NS - 1)
            def _():
                pltpu.sync_copy(acc.at[pl.ds(row0, wpt)],
                                out_hbm.at[c].at[pl.ds(row0, wpt)])

            @pl.when(s == NS - 1)
            def _():
                pltpu.sync_copy(acc.at[pl.ds((NS - 1) * wpt, wlast)],
                                out_hbm.at[c].at[pl.ds((NS - 1) * wpt, wlast)])
        else:
            for h in range(2):
                for k in range(KC):
                    off = s * BPW + k * CWP
                    pltpu.sync_copy(
                        b_hbm.at[pl.ds(pl.multiple_of(h * B + off, 8), CWP)],
                        src_v.at[0])
                    pltpu.async_copy(
                        acc.at[src_v.at[0]], rows_v.at[0], gsem).wait()
                    pltpu.sync_copy(
                        rows_v.at[0],
                        e_hbm.at[h].at[c].at[
                            pl.ds(pl.multiple_of(off, 8), CWP)])

                    @pl.when(c == 0)
                    def _():
                        pltpu.async_copy(
                            nd_hbm.at[src_v.at[0]],
                            ndb_v.at[h * KC + k], gsem).wait()

            @pl.when(c == 0)
            def _():
                pltpu.sync_copy(ndb_v, ndo_hbm.at[s])

    return agg_kernel


def _make_take_kernel(N, D, B):
    BPW = B // NW

    @functools.partial(
        pl.kernel, mesh=_MESH,
        out_type=jax.ShapeDtypeStruct((2, B, D), jnp.float32),
        scratch_types=[
            pltpu.VMEM((BPW,), jnp.int32),
            pltpu.VMEM((BPW, D), jnp.float32),
            pltpu.SemaphoreType.DMA,
        ])
    def take_kernel(v_hbm, b_hbm, out_hbm, bidx_v, rows_v, sem):
        c = lax.axis_index("c")
        s = lax.axis_index("s")
        wid = c * NS + s
        for h in range(2):
            pltpu.sync_copy(
                b_hbm.at[pl.ds(pl.multiple_of(h * B + wid * BPW, 8), BPW)],
                bidx_v)
            pltpu.async_copy(v_hbm.at[bidx_v], rows_v, sem).wait()
            pltpu.sync_copy(
                rows_v,
                out_hbm.at[h].at[pl.ds(pl.multiple_of(wid * BPW, 8), BPW)])

    return take_kernel


def _leaky(x):
    return jnp.where(x > 0, x, 0.01 * x)


def _bn(v, g, bt):
    mu = jnp.mean(v, axis=0, keepdims=True)
    var = jnp.mean((v - mu) ** 2, axis=0, keepdims=True)
    return g * (v - mu) / jnp.sqrt(var + 1e-5) + bt


def _tcpre_body(x_ref, w1_ref, xw_ref):
    xw_ref[...] = jnp.dot(x_ref[...], w1_ref[...],
                          preferred_element_type=jnp.float32)


def _tc1_body(xw_ref, dp_ref, h1_ref, nsnd_ref):
    N = xw_ref.shape[0]
    d = dp_ref[...]
    deg_o = d[:, 0:1] + d[:, 2:3]
    deg_i = d[:, 1:2] + d[:, 3:4]
    ns = lax.rsqrt(jnp.maximum(deg_o, 1.0))
    nd = lax.rsqrt(jnp.maximum(deg_i, 1.0))
    h1_ref[pl.ds(0, N), :] = xw_ref[...] * ns
    h1_ref[pl.ds(N, PADR), :] = jnp.zeros((PADR, h1_ref.shape[1]),
                                          jnp.float32)
    nsnd_ref[...] = jnp.concatenate([ns, nd], axis=1)


def _tc2_body(p_ref, nsnd_ref, b1_ref, g1_ref, bt1_ref, w2_ref, h2_ref):
    N = p_ref.shape[1]
    ns = nsnd_ref[:, 0:1]
    nd = nsnd_ref[:, 1:2]
    v = (p_ref[0] + p_ref[1]) * nd + b1_ref[...]
    v = _leaky(_bn(v, g1_ref[...], bt1_ref[...]))
    h2_ref[pl.ds(0, N), :] = jnp.dot(v * ns, w2_ref[...],
                                     preferred_element_type=jnp.float32)
    h2_ref[pl.ds(N, PADR), :] = jnp.zeros((PADR, h2_ref.shape[1]),
                                          jnp.float32)


def _tc3_body(p_ref, nsnd_ref, b2_ref, v2_ref):
    nd = nsnd_ref[:, 1:2]
    v2_ref[...] = jnp.maximum((p_ref[0] + p_ref[1]) * nd + b2_ref[...], 0.0)


def _tc4_body(e_ref, ndg_ref, b2_ref, f1w_ref, f1b_ref, g2_ref, bt2_ref,
              f2w_ref, f2b_ref, f3w_ref, f3b_ref, out_ref):
    vs = []
    for h in range(2):
        v = ((e_ref[h, 0] + e_ref[h, 1]) * ndg_ref[h][:, None]
             + b2_ref[...])
        vs.append(jnp.maximum(v, 0.0))
    emb = vs[0] - vs[1]
    t = jnp.dot(emb, f1w_ref[...],
                preferred_element_type=jnp.float32) + f1b_ref[...]
    t = _leaky(_bn(t, g2_ref[...], bt2_ref[...]))
    t = _leaky(jnp.dot(t, f2w_ref[...],
                       preferred_element_type=jnp.float32) + f2b_ref[...])
    out_ref[...] = jnp.dot(t, f3w_ref[...],
                           preferred_element_type=jnp.float32) + f3b_ref[...]


def kernel(x, edge_index, batch, W1, b1, W2, b2, g1, bt1, g2, bt2,
           fc1_w, fc1_b, fc2_w, fc2_b, fc3_w, fc3_b):
    N, D = x.shape
    E = edge_index.shape[1]
    B = batch.shape[1]
    H1 = W1.shape[1]
    H2 = fc1_w.shape[0]
    NP = N + PADR
    EPT = E // NW
    CH = -(-EPT // CWP)
    PADE = CH * CWP - EPT
    assert E == NW * EPT and B % NW == 0

    pad = (jnp.arange(PADE, dtype=jnp.int32) % PADR) + N
    padw = jnp.broadcast_to(pad, (NW, PADE))
    src_r = jnp.concatenate(
        [edge_index[0].reshape(NW, EPT), padw], axis=1).reshape(NW, CH, CWP)
    dst_r = jnp.concatenate(
        [edge_index[1].reshape(NW, EPT), padw], axis=1).reshape(NW, CH, CWP)
    zN = jnp.zeros((NP,), jnp.float32)

    xw = pl.pallas_call(
        _tcpre_body,
        out_shape=jax.ShapeDtypeStruct((N, H1), jnp.float32),
    )(x, W1)

    deg = _make_deg_kernel(N, CH)(src_r, dst_r, zN).reshape(NC, 2, NP)
    dp = jnp.transpose(deg[:, :, :N], (2, 0, 1)).reshape(N, 2 * NC)

    h1, nsnd = pl.pallas_call(
        _tc1_body,
        out_shape=(jax.ShapeDtypeStruct((NP, H1), jnp.float32),
                   jax.ShapeDtypeStruct((N, 2), jnp.float32)),
    )(xw, dp)

    agg = _make_agg_kernel(N, H1, CH)
    p1 = agg(h1, src_r, dst_r)

    h2 = pl.pallas_call(
        _tc2_body,
        out_shape=jax.ShapeDtypeStruct((NP, H1), jnp.float32),
    )(p1, nsnd, b1.reshape(1, H1), g1.reshape(1, H1), bt1.reshape(1, H1), W2)

    ndcol = nsnd[:, 1]
    e01, ndo = _make_agg_kernel(N, H1, CH, B=B)(
        h2, src_r, dst_r, batch.reshape(2 * B), ndcol)
    KC = B // NS // CWP
    ndg = jnp.transpose(ndo.reshape(NS, 2, KC, CWP),
                        (1, 0, 2, 3)).reshape(2, B)

    out = pl.pallas_call(
        _tc4_body,
        out_shape=jax.ShapeDtypeStruct((B, 1), jnp.float32),
    )(e01, ndg, b2.reshape(1, H1), fc1_w.T, fc1_b.reshape(1, H2),
      g2.reshape(1, H2), bt2.reshape(1, H2), fc2_w.T, fc2_b.reshape(1, H2),
      fc3_w.T, fc3_b.reshape(1, 1))
    return out


# speedup vs baseline: 1.0245x; 1.0245x over previous
import functools

import jax
import jax.numpy as jnp
from jax import lax
from jax.experimental import pallas as pl
from jax.experimental.pallas import tpu as pltpu
from jax.experimental.pallas import tpu_sc as plsc

NC, NS = 2, 16
NW = NC * NS
CWP = 128
PADR = 16

_MESH = plsc.VectorSubcoreMesh(
    core_axis_name="c", subcore_axis_name="s", num_cores=NC, num_subcores=NS)


def _make_deg_kernel(N, CH):
    NP = N + PADR

    @functools.partial(
        pl.kernel, mesh=_MESH,
        out_type=jax.ShapeDtypeStruct((NC * 2 * NP,), jnp.float32),
        scratch_types=[
            pltpu.VMEM((CH, CWP), jnp.int32),
            pltpu.VMEM((CH, CWP), jnp.int32),
            pltpu.VMEM((CWP,), jnp.float32),
            pltpu.VMEM((NP,), jnp.float32),
            pltpu.VMEM_SHARED((NP,), jnp.float32),
            pltpu.VMEM_SHARED((NP,), jnp.float32),
            pltpu.SemaphoreType.DMA,
        ])
    def deg_kernel(src_hbm, dst_hbm, z_hbm, out_hbm,
                   src_v, dst_v, ones_v, tmp_v, acc_o, acc_i, sem):
        c = lax.axis_index("c")
        s = lax.axis_index("s")
        wid = c * NS + s
        pltpu.sync_copy(src_hbm.at[wid], src_v)
        pltpu.sync_copy(dst_hbm.at[wid], dst_v)
        for i in range(CWP // 16):
            ones_v[pl.ds(i * 16, 16)] = jnp.full((16,), 1.0, jnp.float32)

        @pl.when(s == 0)
        def _():
            pltpu.sync_copy(z_hbm, acc_o)

        @pl.when(s == 1)
        def _():
            pltpu.sync_copy(z_hbm, acc_i)

        plsc.subcore_barrier()

        LAG = 4

        def body(j, carry):
            @pl.when(j >= LAG)
            def _():
                pltpu.make_async_copy(
                    ones_v, acc_o.at[src_v.at[0]], sem).wait()
                pltpu.make_async_copy(
                    ones_v, acc_i.at[dst_v.at[0]], sem).wait()

            pltpu.async_copy(ones_v, acc_o.at[src_v.at[j]], sem, add=True)
            pltpu.async_copy(ones_v, acc_i.at[dst_v.at[j]], sem, add=True)
            return carry

        lax.fori_loop(0, CH, body, 0)
        for _ in range(LAG):
            pltpu.make_async_copy(ones_v, acc_o.at[src_v.at[0]], sem).wait()
            pltpu.make_async_copy(ones_v, acc_i.at[dst_v.at[0]], sem).wait()
        plsc.subcore_barrier()

        @pl.when(s == 0)
        def _():
            pltpu.sync_copy(acc_o, tmp_v)
            pltpu.sync_copy(
                tmp_v, out_hbm.at[pl.ds(pl.multiple_of(c * 2 * NP, 8), NP)])

        @pl.when(s == 1)
        def _():
            pltpu.sync_copy(acc_i, tmp_v)
            pltpu.sync_copy(
                tmp_v,
                out_hbm.at[pl.ds(pl.multiple_of(c * 2 * NP + NP, 8), NP)])

    return deg_kernel


def _make_agg_kernel(N, D, CH, B=None):
    NP = N + PADR

    PH = (CH + 1) // 2
    if B is None:
        out_type = jax.ShapeDtypeStruct((NC, N, D), jnp.float32)
        extra_scr = []
    else:
        BPW = B // NS
        KC = BPW // CWP
        out_type = (jax.ShapeDtypeStruct((2, NC, B, D), jnp.float32),
                    jax.ShapeDtypeStruct((NS, 2 * KC, CWP), jnp.float32))
        extra_scr = [pltpu.VMEM((2 * KC, CWP), jnp.float32),
                     pltpu.SemaphoreType.DMA]

    @functools.partial(
        pl.kernel, mesh=_MESH,
        out_type=out_type,
        scratch_types=[
            pltpu.VMEM((PH, CWP), jnp.int32),
            pltpu.VMEM((PH, CWP), jnp.int32),
            pltpu.VMEM((2, CWP, D), jnp.float32),
            pltpu.VMEM((16, D), jnp.float32),
            pltpu.VMEM_SHARED((NP, D), jnp.float32),
            pltpu.SemaphoreType.DMA,
            pltpu.SemaphoreType.DMA,
        ] + extra_scr)
    def agg_kernel(h_hbm, src_hbm, dst_hbm, *rest):
        if B is None:
            (out_hbm, src_v, dst_v, rows_v, zb_v, acc, gsem, ssem) = rest
        else:
            (b_hbm, nd_hbm, e_hbm, ndo_hbm,
             src_v, dst_v, rows_v, zb_v, acc, gsem, ssem, ndb_v,
             ndsem) = rest
        c = lax.axis_index("c")
        s = lax.axis_index("s")
        wid = c * NS + s

        def zrow(i, carry):
            for jj in range(D // 16):
                zb_v[i, pl.ds(jj * 16, 16)] = jnp.zeros((16,), jnp.float32)
            return carry

        lax.fori_loop(0, 16, zrow, 0)

        rpt = (NP // NS) & ~7
        last = NP - (NS - 1) * rpt
        nz = rpt // 16 + jnp.where(s == NS - 1, (last - rpt) // 16, 0)

        def zcopy(i, carry):
            r0 = pl.multiple_of(s * rpt + i * 16, 8)
            pltpu.sync_copy(zb_v, acc.at[pl.ds(r0, 16)])
            return carry

        lax.fori_loop(0, nz, zcopy, 0)
        plsc.subcore_barrier()

        def phase(base, nj):
            pltpu.sync_copy(src_hbm.at[wid].at[pl.ds(base, nj)],
                            src_v.at[pl.ds(0, nj)])
            pltpu.sync_copy(dst_hbm.at[wid].at[pl.ds(base, nj)],
                            dst_v.at[pl.ds(0, nj)])
            pltpu.async_copy(h_hbm.at[src_v.at[0]], rows_v.at[0], gsem)

            def body(j, carry):
                cur = lax.rem(j, 2)

                @pl.when(j >= 1)
                def _():
                    pltpu.make_async_copy(
                        rows_v.at[1 - cur], acc.at[dst_v.at[0]], ssem).wait()

                @pl.when(j + 1 < nj)
                def _():
                    pltpu.async_copy(
                        h_hbm.at[src_v.at[j + 1]], rows_v.at[1 - cur], gsem)

                pltpu.make_async_copy(
                    h_hbm.at[src_v.at[j]], rows_v.at[cur], gsem).wait()
                pltpu.async_copy(rows_v.at[cur], acc.at[dst_v.at[j]], ssem,
                                 add=True)
                return carry

            lax.fori_loop(0, nj, body, 0)
            pltpu.make_async_copy(
                rows_v.at[0], acc.at[dst_v.at[0]], ssem).wait()

        phase(0, PH)
        phase(PH, CH - PH)
        plsc.subcore_barrier()

        if B is None:
            wpt = (N // NS) & ~7
            wlast = N - (NS - 1) * wpt
            row0 = pl.multiple_of(s * wpt, 8)

            @pl.when(s < NS - 1)
            def _():
                pltpu.sync_copy(acc.at[pl.ds(row0, wpt)],
                                out_hbm.at[c].at[pl.ds(row0, wpt)])

            @pl.when(s == NS - 1)
            def _():
                pltpu.sync_copy(acc.at[pl.ds((NS - 1) * wpt, wlast)],
                                out_hbm.at[c].at[pl.ds((NS - 1) * wpt, wlast)])
        else:
            chunks = [(h, k) for h in range(2) for k in range(KC)]
            for i, (h, k) in enumerate(chunks):
                q = i % 2
                off = s * BPW + k * CWP
                if i >= 2:
                    @pl.when(c == 0)
                    def _():
                        pltpu.make_async_copy(
                            nd_hbm.at[src_v.at[q]], ndb_v.at[0],
                            ndsem).wait()

                pltpu.sync_copy(
                    b_hbm.at[pl.ds(pl.multiple_of(h * B + off, 8), CWP)],
                    src_v.at[q])
                pltpu.async_copy(
                    acc.at[src_v.at[q]], rows_v.at[q], gsem).wait()

                @pl.when(c == 0)
                def _():
                    pltpu.async_copy(nd_hbm.at[src_v.at[q]],
                                     ndb_v.at[h * KC + k], ndsem)

                if i >= 2:
                    pltpu.make_async_copy(
                        rows_v.at[q],
                        e_hbm.at[0].at[c].at[pl.ds(0, CWP)], ssem).wait()
                pltpu.async_copy(
                    rows_v.at[q],
                    e_hbm.at[h].at[c].at[pl.ds(pl.multiple_of(off, 8), CWP)],
                    ssem)
            for q in (0, 1):
                pltpu.make_async_copy(
                    rows_v.at[q],
                    e_hbm.at[0].at[c].at[pl.ds(0, CWP)], ssem).wait()

            @pl.when(c == 0)
            def _():
                pltpu.make_async_copy(
                    nd_hbm.at[src_v.at[0]], ndb_v.at[0], ndsem).wait()
                pltpu.make_async_copy(
                    nd_hbm.at[src_v.at[0]], ndb_v.at[0], ndsem).wait()
                pltpu.sync_copy(ndb_v, ndo_hbm.at[s])

    return agg_kernel


def _make_take_kernel(N, D, B):
    BPW = B // NW

    @functools.partial(
        pl.kernel, mesh=_MESH,
        out_type=jax.ShapeDtypeStruct((2, B, D), jnp.float32),
        scratch_types=[
            pltpu.VMEM((BPW,), jnp.int32),
            pltpu.VMEM((BPW, D), jnp.float32),
            pltpu.SemaphoreType.DMA,
        ])
    def take_kernel(v_hbm, b_hbm, out_hbm, bidx_v, rows_v, sem):
        c = lax.axis_index("c")
        s = lax.axis_index("s")
        wid = c * NS + s
        for h in range(2):
            pltpu.sync_copy(
                b_hbm.at[pl.ds(pl.multiple_of(h * B + wid * BPW, 8), BPW)],
                bidx_v)
            pltpu.async_copy(v_hbm.at[bidx_v], rows_v, sem).wait()
            pltpu.sync_copy(
                rows_v,
                out_hbm.at[h].at[pl.ds(pl.multiple_of(wid * BPW, 8), BPW)])

    return take_kernel


def _leaky(x):
    return jnp.where(x > 0, x, 0.01 * x)


def _bn(v, g, bt):
    mu = jnp.mean(v, axis=0, keepdims=True)
    var = jnp.mean((v - mu) ** 2, axis=0, keepdims=True)
    return g * (v - mu) / jnp.sqrt(var + 1e-5) + bt


def _tc1_body(x_ref, dp_ref, w1_ref, h1_ref, nsnd_ref):
    N = x_ref.shape[0]
    d = dp_ref[...]
    deg_o = d[:, 0:1] + d[:, 2:3]
    deg_i = d[:, 1:2] + d[:, 3:4]
    ns = lax.rsqrt(jnp.maximum(deg_o, 1.0))
    nd = lax.rsqrt(jnp.maximum(deg_i, 1.0))
    h1_ref[pl.ds(0, N), :] = jnp.dot(x_ref[...] * ns, w1_ref[...],
                                     preferred_element_type=jnp.float32)
    h1_ref[pl.ds(N, PADR), :] = jnp.zeros((PADR, h1_ref.shape[1]),
                                          jnp.float32)
    nsnd_ref[...] = jnp.concatenate([ns, nd], axis=1)


def _tc2_body(p_ref, nsnd_ref, b1_ref, g1_ref, bt1_ref, w2_ref, h2_ref):
    N = p_ref.shape[1]
    ns = nsnd_ref[:, 0:1]
    nd = nsnd_ref[:, 1:2]
    v = (p_ref[0] + p_ref[1]) * nd + b1_ref[...]
    v = _leaky(_bn(v, g1_ref[...], bt1_ref[...]))
    h2_ref[pl.ds(0, N), :] = jnp.dot(v * ns, w2_ref[...],
                                     preferred_element_type=jnp.float32)
    h2_ref[pl.ds(N, PADR), :] = jnp.zeros((PADR, h2_ref.shape[1]),
                                          jnp.float32)


def _tc3_body(p_ref, nsnd_ref, b2_ref, v2_ref):
    nd = nsnd_ref[:, 1:2]
    v2_ref[...] = jnp.maximum((p_ref[0] + p_ref[1]) * nd + b2_ref[...], 0.0)


def _tc4_body(e_ref, ndg_ref, b2_ref, f1w_ref, f1b_ref, g2_ref, bt2_ref,
              f2w_ref, f2b_ref, f3w_ref, f3b_ref, out_ref):
    vs = []
    for h in range(2):
        v = ((e_ref[h, 0] + e_ref[h, 1]) * ndg_ref[h][:, None]
             + b2_ref[...])
        vs.append(jnp.maximum(v, 0.0))
    emb = vs[0] - vs[1]
    t = jnp.dot(emb, f1w_ref[...],
                preferred_element_type=jnp.float32) + f1b_ref[...]
    t = _leaky(_bn(t, g2_ref[...], bt2_ref[...]))
    t = _leaky(jnp.dot(t, f2w_ref[...],
                       preferred_element_type=jnp.float32) + f2b_ref[...])
    out_ref[...] = jnp.dot(t, f3w_ref[...],
                           preferred_element_type=jnp.float32) + f3b_ref[...]


def kernel(x, edge_index, batch, W1, b1, W2, b2, g1, bt1, g2, bt2,
           fc1_w, fc1_b, fc2_w, fc2_b, fc3_w, fc3_b):
    N, D = x.shape
    E = edge_index.shape[1]
    B = batch.shape[1]
    H1 = W1.shape[1]
    H2 = fc1_w.shape[0]
    NP = N + PADR
    EPT = E // NW
    CH = -(-EPT // CWP)
    PADE = CH * CWP - EPT
    assert E == NW * EPT and B % NW == 0

    pad = (jnp.arange(PADE, dtype=jnp.int32) % PADR) + N
    padw = jnp.broadcast_to(pad, (NW, PADE))
    src_r = jnp.concatenate(
        [edge_index[0].reshape(NW, EPT), padw], axis=1).reshape(NW, CH, CWP)
    dst_r = jnp.concatenate(
        [edge_index[1].reshape(NW, EPT), padw], axis=1).reshape(NW, CH, CWP)
    zN = jnp.zeros((NP,), jnp.float32)

    deg = _make_deg_kernel(N, CH)(src_r, dst_r, zN).reshape(NC, 2, NP)
    dp = jnp.transpose(deg[:, :, :N], (2, 0, 1)).reshape(N, 2 * NC)

    h1, nsnd = pl.pallas_call(
        _tc1_body,
        out_shape=(jax.ShapeDtypeStruct((NP, H1), jnp.float32),
                   jax.ShapeDtypeStruct((N, 2), jnp.float32)),
    )(x, dp, W1)

    agg = _make_agg_kernel(N, H1, CH)
    p1 = agg(h1, src_r, dst_r)

    h2 = pl.pallas_call(
        _tc2_body,
        out_shape=jax.ShapeDtypeStruct((NP, H1), jnp.float32),
    )(p1, nsnd, b1.reshape(1, H1), g1.reshape(1, H1), bt1.reshape(1, H1), W2)

    ndcol = nsnd[:, 1]
    e01, ndo = _make_agg_kernel(N, H1, CH, B=B)(
        h2, src_r, dst_r, batch.reshape(2 * B), ndcol)
    KC = B // NS // CWP
    ndg = jnp.transpose(ndo.reshape(NS, 2, KC, CWP),
                        (1, 0, 2, 3)).reshape(2, B)

    out = pl.pallas_call(
        _tc4_body,
        out_shape=jax.ShapeDtypeStruct((B, 1), jnp.float32),
    )(e01, ndg, b2.reshape(1, H1), fc1_w.T, fc1_b.reshape(1, H2),
      g2.reshape(1, H2), bt2.reshape(1, H2), fc2_w.T, fc2_b.reshape(1, H2),
      fc3_w.T, fc3_b.reshape(1, 1))
    return out
